# baseline (speedup 1.0000x reference)
import functools

import jax
import jax.numpy as jnp
from jax import lax
from jax.experimental import pallas as pl
from jax.experimental.pallas import tpu as pltpu
from jax.experimental.pallas import tpu_sc as plsc

_N = 9870
_D = 128
_H = 128
_B = 141
_E = 157920

_NW = 32
_CHUNK = 128
_NCH = 40
_STRIPE = 624
_ACC = 15 * _STRIPE + 512
_NPAD = 9984

_R = 1248
_NB = _NPAD // _R

_HI = lax.Precision.HIGHEST


def _dot(a, b):
    return jnp.dot(a, b, preferred_element_type=jnp.float32, precision=_HI)


def _dot_d(a, b):
    return jnp.dot(a, b, preferred_element_type=jnp.float32)


def _onehot(lo, hi, j):
    r = (j * _R + lax.broadcasted_iota(jnp.int32, (_R, _B), 0))
    return ((r >= lo) & (r < hi)).astype(jnp.float32)


def _softmax(al):
    m = jnp.max(al, axis=-1, keepdims=True)
    e = jnp.exp(al - m)
    return e / jnp.sum(e, axis=-1, keepdims=True)


def _gnew_of(out, g, glbW, glbb):
    z = jnp.concatenate([out, g], axis=1)
    return g + jnp.tanh(_dot_d(z, glbW) + glbb)


def _relu_rows(aggh_ref, aggg_ref, q_ref, bl_ref, wlt, wlb):
    agg_h = aggh_ref[0] + aggh_ref[1]
    h = jnp.maximum(_dot_d(agg_h, wlt) + _dot_d(aggg_ref[...], wlb)
                    + bl_ref[...] + q_ref[...], 0.0)
    r = (pl.program_id(0) * _R
         + lax.broadcasted_iota(jnp.int32, (_R, 1), 0))
    return jnp.where(r < _N, h, 0.0)


def _p1_body(first, last):
    def body(*refs):
        if first:
            (x_ref, lo_ref, hi_ref, attW, attb, featW, featb, out_ref) = refs
            h = x_ref[...]
        elif last:
            (aggh_ref, aggg_ref, q_ref, bl_ref, lo_ref, hi_ref, g_ref,
             wlt, wlb, attW, attb, featW, featb, glbW, glbb,
             h_out, out_ref, g_out) = refs
            h = _relu_rows(aggh_ref, aggg_ref, q_ref, bl_ref, wlt[...],
                           wlb[...])
            h_out[...] = h
        else:
            (aggh_ref, aggg_ref, q_ref, bl_ref, lo_ref, hi_ref,
             wlt, wlb, attW, attb, featW, featb, h_out, out_ref) = refs
            h = _relu_rows(aggh_ref, aggg_ref, q_ref, bl_ref, wlt[...],
                           wlb[...])
            h_out[...] = h
        j = pl.program_id(0)
        a = _softmax(_dot_d(h, attW[...]) + attb[...])
        f = _dot_d(h, featW[...]) + featb[...]
        prod = a * f
        oh = _onehot(lo_ref[...], hi_ref[...], j)
        contrib = lax.dot_general(oh, prod, (((0,), (0,)), ((), ())),
                                  preferred_element_type=jnp.float32)

        @pl.when(j == 0)
        def _():
            out_ref[...] = jnp.zeros_like(out_ref)

        out_ref[...] += contrib
        if last:
            @pl.when(j == _NB - 1)
            def _():
                g_out[...] = _gnew_of(out_ref[...], g_ref[...], glbW[...],
                                      glbb[...])
    return body


def _p3p_body(out_ref, g_ref, glbW, glbb, g_out):
    g_out[...] = _gnew_of(out_ref[...], g_ref[...], glbW[...], glbb[...])


def _p3q_body(h_ref, lo_ref, hi_ref, g_ref, wr, Ca_ref, Cb_ref, q_out,
              aggg_out):
    j = pl.program_id(0)
    oh = _onehot(lo_ref[...], hi_ref[...], j)
    g = g_ref[...]
    grep = _dot(oh, g)
    h2 = jnp.concatenate([h_ref[...], grep], axis=1)
    q_out[...] = _dot_d(h2, wr[...])
    ga = g[0:128, :]
    gb = jnp.concatenate(
        [g[128:_B, :], jnp.zeros((128 - (_B - 128), _H), jnp.float32)],
        axis=0)
    aggg_out[...] = _dot(Ca_ref[...], ga) + _dot(Cb_ref[...], gb)


def _csum_body(cp_ref, c_out):
    c_out[...] = cp_ref[0] + cp_ref[1]


_row = pl.BlockSpec((_R, _H), lambda j: (j, 0))
_full = lambda r, c: pl.BlockSpec((r, c), lambda j: (0, 0))
_segb = pl.BlockSpec((1, _B), lambda j: (0, 0))
_aggb = pl.BlockSpec((2, _R, _H), lambda j: (0, j, 0))
_cb = pl.BlockSpec((_R, _H), lambda j: (j, 0))
_cpb = pl.BlockSpec((2, _R, _H), lambda j: (0, j, 0))
_w = _full(_H, _H)
_w2 = _full(2 * _H, _H)
_b1 = _full(1, _H)
_gB = _full(_B, _H)

_f32 = jnp.float32


def _p1_first():
    return pl.pallas_call(
        _p1_body(True, False),
        grid=(_NB,),
        in_specs=[_row, _segb, _segb, _w, _b1, _w, _b1],
        out_specs=_gB,
        out_shape=jax.ShapeDtypeStruct((_B, _H), _f32),
    )


def _p1_mid():
    return pl.pallas_call(
        _p1_body(False, False),
        grid=(_NB,),
        in_specs=[_aggb, _row, _row, _b1, _segb, _segb, _w, _w, _w, _b1,
                  _w, _b1],
        out_specs=(_row, _gB),
        out_shape=(jax.ShapeDtypeStruct((_NPAD, _H), _f32),
                   jax.ShapeDtypeStruct((_B, _H), _f32)),
    )


def _p1_last():
    return pl.pallas_call(
        _p1_body(False, True),
        grid=(_NB,),
        in_specs=[_aggb, _row, _row, _b1, _segb, _segb, _gB, _w, _w,
                  _w, _b1, _w, _b1, _w2, _b1],
        out_specs=(_row, _gB, _gB),
        out_shape=(jax.ShapeDtypeStruct((_NPAD, _H), _f32),
                   jax.ShapeDtypeStruct((_B, _H), _f32),
                   jax.ShapeDtypeStruct((_B, _H), _f32)),
    )


def _p3p():
    return pl.pallas_call(
        _p3p_body,
        out_shape=jax.ShapeDtypeStruct((_B, _H), _f32),
    )


def _p3q():
    return pl.pallas_call(
        _p3q_body,
        grid=(_NB,),
        in_specs=[_row, _segb, _segb, _gB, _w2, _cb, _cb],
        out_specs=(_row, _row),
        out_shape=(jax.ShapeDtypeStruct((_NPAD, _H), _f32),
                   jax.ShapeDtypeStruct((_NPAD, _H), _f32)),
    )


def _csum():
    return pl.pallas_call(
        _csum_body,
        grid=(_NB,),
        in_specs=[_cpb],
        out_specs=_cb,
        out_shape=jax.ShapeDtypeStruct((_ACC, _H), _f32),
    )


def _sc_scatter(width, chunk, nch):
    @functools.partial(
        pl.kernel,
        out_type=jax.ShapeDtypeStruct((2, _ACC, width), jnp.float32),
        mesh=plsc.VectorSubcoreMesh(core_axis_name="c", subcore_axis_name="s"),
        scratch_types=[
            pltpu.VMEM((nch, chunk), jnp.int32),
            pltpu.VMEM((nch, chunk), jnp.int32),
            pltpu.VMEM((2, chunk, width), jnp.float32),
            pltpu.VMEM_SHARED((_ACC, width), jnp.float32),
            pltpu.SemaphoreType.DMA,
            pltpu.SemaphoreType.DMA,
        ],
    )
    def k(tab_hbm, src_hbm, dst_hbm, zer_hbm, out_hbm, src_v, dst_v, rows,
          acc_sh, sg0, sg1):
        sem_g = [sg0, sg1]
        c = lax.axis_index("c")
        s = lax.axis_index("s")
        wid = s * 2 + c

        @pl.when(s < 15)
        def _():
            pltpu.sync_copy(zer_hbm, acc_sh.at[pl.ds(s * _STRIPE, _STRIPE)])

        @pl.when(s == 15)
        def _():
            pltpu.sync_copy(zer_hbm.at[pl.ds(0, 512)],
                            acc_sh.at[pl.ds(15 * _STRIPE, 512)])

        pltpu.sync_copy(src_hbm.at[wid], src_v)
        pltpu.sync_copy(dst_hbm.at[wid], dst_v)
        plsc.subcore_barrier()

        pltpu.async_copy(tab_hbm.at[src_v.at[0]], rows.at[0], sem_g[0])
        pltpu.async_copy(tab_hbm.at[src_v.at[1]], rows.at[1], sem_g[1])

        def body(t, carry):
            j0 = 2 * t
            for k2 in range(2):
                pltpu.make_async_copy(tab_hbm.at[src_v.at[j0 + k2]],
                                      rows.at[k2], sem_g[k2]).wait()
                pltpu.sync_copy(rows.at[k2], acc_sh.at[dst_v.at[j0 + k2]],
                                add=True)

                @pl.when(j0 + 2 + k2 < nch)
                def _():
                    pltpu.async_copy(tab_hbm.at[src_v.at[j0 + 2 + k2]],
                                     rows.at[k2], sem_g[k2])
            return carry

        lax.fori_loop(0, nch // 2, body, 0)
        plsc.subcore_barrier()

        @pl.when(s < 15)
        def _():
            pltpu.sync_copy(acc_sh.at[pl.ds(s * _STRIPE, _STRIPE)],
                            out_hbm.at[c].at[pl.ds(s * _STRIPE, _STRIPE)])

        @pl.when(s == 15)
        def _():
            pltpu.sync_copy(acc_sh.at[pl.ds(15 * _STRIPE, 512)],
                            out_hbm.at[c].at[pl.ds(15 * _STRIPE, 512)])

    return k


def _pad_idx(idx, fill, nch, chunk):
    pad = _NW * nch * chunk - _E
    fidx = jnp.arange(pad, dtype=jnp.int32)
    return jnp.concatenate([idx, fill(fidx)]).reshape(_NW, nch, chunk)


def kernel(x, edge_index, batch_sizes, att_W, att_b, feat_W, feat_b,
           glb_W, glb_b, sage_Wl, sage_bl, sage_Wr):
    offs = jnp.concatenate([jnp.zeros((1,), jnp.int32),
                            jnp.cumsum(batch_sizes, dtype=jnp.int32)])
    lo = offs[:_B].reshape(1, _B)
    hi = offs[1:].reshape(1, _B)
    x_pad = jnp.concatenate(
        [x, jnp.zeros((_NPAD - _N, _D), jnp.float32)], axis=0)
    src = edge_index[0]
    dst = edge_index[1]
    gsrc = jnp.searchsorted(offs[1:], src, side='right').astype(jnp.int32)
    src_p = _pad_idx(src, lambda f: f % _N, _NCH, _CHUNK)
    dst_p = _pad_idx(dst, lambda f: _N + f % (_ACC - _N), _NCH, _CHUNK)
    gsrc_p = _pad_idx(gsrc, lambda f: jnp.zeros_like(f), _NCH, _CHUNK)
    zer = jnp.zeros((_STRIPE, _H), jnp.float32)
    eye = jnp.eye(128, dtype=jnp.float32)
    eyeA = jnp.concatenate([eye, jnp.zeros((16, 128), jnp.float32)], axis=0)
    eyeB = jnp.concatenate([jnp.zeros((128, 128), jnp.float32),
                            eye[: _B - 128],
                            jnp.zeros((144 - _B, 128), jnp.float32)], axis=0)
    g = jnp.zeros((_B, _H), jnp.float32)

    aw = lambda i: att_W[i]
    ab = lambda i: att_b[i].reshape(1, _H)
    fw = lambda i: feat_W[i]
    fb = lambda i: feat_b[i].reshape(1, _H)
    gw = lambda i: glb_W[i]
    gb = lambda i: glb_b[i].reshape(1, _H)
    wlt = lambda i: sage_Wl[i, :_D]
    wlb = lambda i: sage_Wl[i, _D:]
    wr = lambda i: sage_Wr[i]
    blr = lambda i: sage_bl[i].reshape(1, _H)

    cpa = _sc_scatter(_H, _CHUNK, _NCH)(eyeA, gsrc_p, dst_p, zer)
    cpb = _sc_scatter(_H, _CHUNK, _NCH)(eyeB, gsrc_p, dst_p, zer)
    Ca = _csum()(cpa)
    Cb = _csum()(cpb)
    aggs = _sc_scatter(_H, _CHUNK, _NCH)(x_pad, src_p, dst_p, zer)
    out = _p1_first()(x_pad, lo, hi, aw(0), ab(0), fw(0), fb(0))
    g = _p3p()(out, g, gw(0), gb(0))
    q, aggg = _p3q()(x_pad, lo, hi, g, wr(0), Ca, Cb)
    for i in (1, 2):
        h, out = _p1_mid()(aggs, aggg, q, blr(i - 1), lo, hi,
                           wlt(i - 1), wlb(i - 1), aw(i), ab(i), fw(i),
                           fb(i))
        aggs = _sc_scatter(_H, _CHUNK, _NCH)(h, src_p, dst_p, zer)
        g = _p3p()(out, g, gw(i), gb(i))
        q, aggg = _p3q()(h, lo, hi, g, wr(i), Ca, Cb)
    h, _, g = _p1_last()(aggs, aggg, q, blr(2), lo, hi, g, wlt(2), wlb(2),
                         aw(3), ab(3), fw(3), fb(3), gw(3), gb(3))
    return (h[:_N], g)


# speedup vs baseline: 5.0517x; 5.0517x over previous
import functools

import jax
import jax.numpy as jnp
from jax import lax
from jax.experimental import pallas as pl
from jax.experimental.pallas import tpu as pltpu
from jax.experimental.pallas import tpu_sc as plsc

_N = 9870
_D = 128
_H = 128
_B = 141
_E = 157920

_NW = 32
_CHUNK = 128
_NCH = 40
_STRIPE = 624
_ACC = 15 * _STRIPE + 512
_NPAD = 9984

_R = 1248
_NB = _NPAD // _R

_HI = lax.Precision.HIGHEST


def _dot(a, b):
    return jnp.dot(a, b, preferred_element_type=jnp.float32, precision=_HI)


def _dot_d(a, b):
    return jnp.dot(a, b, preferred_element_type=jnp.float32)


def _onehot(lo, hi, j):
    r = (j * _R + lax.broadcasted_iota(jnp.int32, (_R, _B), 0))
    return ((r >= lo) & (r < hi)).astype(jnp.float32)


def _softmax(al):
    m = jnp.max(al, axis=-1, keepdims=True)
    e = jnp.exp(al - m)
    return e / jnp.sum(e, axis=-1, keepdims=True)


def _gnew_of(out, g, glbW, glbb):
    z = jnp.concatenate([out, g], axis=1)
    return g + jnp.tanh(_dot_d(z, glbW) + glbb)


def _relu_rows(aggh_ref, aggg_ref, q_ref, bl_ref, wlt, wlb):
    agg_h = aggh_ref[0] + aggh_ref[1]
    h = jnp.maximum(_dot_d(agg_h, wlt) + _dot_d(aggg_ref[...], wlb)
                    + bl_ref[...] + q_ref[...], 0.0)
    r = (pl.program_id(0) * _R
         + lax.broadcasted_iota(jnp.int32, (_R, 1), 0))
    return jnp.where(r < _N, h, 0.0)


def _p1_body(first, last):
    def body(*refs):
        if first:
            (x_ref, lo_ref, hi_ref, attW, attb, featW, featb, out_ref) = refs
            h = x_ref[...]
        elif last:
            (aggh_ref, aggg_ref, q_ref, bl_ref, lo_ref, hi_ref, g_ref,
             wlt, wlb, attW, attb, featW, featb, glbW, glbb,
             h_out, out_ref, g_out) = refs
            h = _relu_rows(aggh_ref, aggg_ref, q_ref, bl_ref, wlt[...],
                           wlb[...])
            h_out[...] = h
        else:
            (aggh_ref, aggg_ref, q_ref, bl_ref, lo_ref, hi_ref,
             wlt, wlb, attW, attb, featW, featb, h_out, out_ref) = refs
            h = _relu_rows(aggh_ref, aggg_ref, q_ref, bl_ref, wlt[...],
                           wlb[...])
            h_out[...] = h
        j = pl.program_id(0)
        a = _softmax(_dot_d(h, attW[...]) + attb[...])
        f = _dot_d(h, featW[...]) + featb[...]
        prod = a * f
        oh = _onehot(lo_ref[...], hi_ref[...], j)
        contrib = lax.dot_general(oh, prod, (((0,), (0,)), ((), ())),
                                  preferred_element_type=jnp.float32)

        @pl.when(j == 0)
        def _():
            out_ref[...] = jnp.zeros_like(out_ref)

        out_ref[...] += contrib
        if last:
            @pl.when(j == _NB - 1)
            def _():
                g_out[...] = _gnew_of(out_ref[...], g_ref[...], glbW[...],
                                      glbb[...])
    return body


def _p3p_body(out_ref, g_ref, glbW, glbb, g_out):
    g_out[...] = _gnew_of(out_ref[...], g_ref[...], glbW[...], glbb[...])


def _p3q_body(h_ref, lo_ref, hi_ref, g_ref, wr, Ca_ref, Cb_ref, q_out,
              aggg_out):
    j = pl.program_id(0)
    oh = _onehot(lo_ref[...], hi_ref[...], j)
    g = g_ref[...]
    grep = _dot(oh, g)
    h2 = jnp.concatenate([h_ref[...], grep], axis=1)
    q_out[...] = _dot_d(h2, wr[...])
    ga = g[0:128, :]
    gb = jnp.concatenate(
        [g[128:_B, :], jnp.zeros((128 - (_B - 128), _H), jnp.float32)],
        axis=0)
    aggg_out[...] = _dot(Ca_ref[...], ga) + _dot(Cb_ref[...], gb)


def _csum_body(cp_ref, c_out):
    c_out[...] = cp_ref[0] + cp_ref[1]


_row = pl.BlockSpec((_R, _H), lambda j: (j, 0))
_full = lambda r, c: pl.BlockSpec((r, c), lambda j: (0, 0))
_segb = pl.BlockSpec((1, _B), lambda j: (0, 0))
_aggb = pl.BlockSpec((2, _R, _H), lambda j: (0, j, 0))
_cb = pl.BlockSpec((_R, _H), lambda j: (j, 0))
_cpb = pl.BlockSpec((2, _R, _H), lambda j: (0, j, 0))
_w = _full(_H, _H)
_w2 = _full(2 * _H, _H)
_b1 = _full(1, _H)
_gB = _full(_B, _H)

_f32 = jnp.float32


def _p1_first():
    return pl.pallas_call(
        _p1_body(True, False),
        grid=(_NB,),
        in_specs=[_row, _segb, _segb, _w, _b1, _w, _b1],
        out_specs=_gB,
        out_shape=jax.ShapeDtypeStruct((_B, _H), _f32),
    )


def _p1_mid():
    return pl.pallas_call(
        _p1_body(False, False),
        grid=(_NB,),
        in_specs=[_aggb, _row, _row, _b1, _segb, _segb, _w, _w, _w, _b1,
                  _w, _b1],
        out_specs=(_row, _gB),
        out_shape=(jax.ShapeDtypeStruct((_NPAD, _H), _f32),
                   jax.ShapeDtypeStruct((_B, _H), _f32)),
    )


def _p1_last():
    return pl.pallas_call(
        _p1_body(False, True),
        grid=(_NB,),
        in_specs=[_aggb, _row, _row, _b1, _segb, _segb, _gB, _w, _w,
                  _w, _b1, _w, _b1, _w2, _b1],
        out_specs=(_row, _gB, _gB),
        out_shape=(jax.ShapeDtypeStruct((_NPAD, _H), _f32),
                   jax.ShapeDtypeStruct((_B, _H), _f32),
                   jax.ShapeDtypeStruct((_B, _H), _f32)),
    )


def _p3p():
    return pl.pallas_call(
        _p3p_body,
        out_shape=jax.ShapeDtypeStruct((_B, _H), _f32),
    )


def _p3q():
    return pl.pallas_call(
        _p3q_body,
        grid=(_NB,),
        in_specs=[_row, _segb, _segb, _gB, _w2, _cb, _cb],
        out_specs=(_row, _row),
        out_shape=(jax.ShapeDtypeStruct((_NPAD, _H), _f32),
                   jax.ShapeDtypeStruct((_NPAD, _H), _f32)),
    )


def _csum():
    return pl.pallas_call(
        _csum_body,
        grid=(_NB,),
        in_specs=[_cpb],
        out_specs=_cb,
        out_shape=jax.ShapeDtypeStruct((_ACC, _H), _f32),
    )


def _sc_scatter(width, chunk, nch):
    @functools.partial(
        pl.kernel,
        out_type=jax.ShapeDtypeStruct((2, _ACC, width), jnp.float32),
        mesh=plsc.VectorSubcoreMesh(core_axis_name="c", subcore_axis_name="s"),
        scratch_types=[
            pltpu.VMEM((nch, chunk), jnp.int32),
            pltpu.VMEM((nch, chunk), jnp.int32),
            pltpu.VMEM((2, chunk, width), jnp.float32),
            pltpu.VMEM_SHARED((_ACC, width), jnp.float32),
            pltpu.SemaphoreType.DMA,
            pltpu.SemaphoreType.DMA,
        ],
    )
    def k(tab_hbm, src_hbm, dst_hbm, zer_hbm, out_hbm, src_v, dst_v, rows,
          acc_sh, sg0, sg1):
        sem_g = [sg0, sg1]
        c = lax.axis_index("c")
        s = lax.axis_index("s")
        wid = s * 2 + c

        @pl.when(s < 15)
        def _():
            pltpu.sync_copy(zer_hbm, acc_sh.at[pl.ds(s * _STRIPE, _STRIPE)])

        @pl.when(s == 15)
        def _():
            pltpu.sync_copy(zer_hbm.at[pl.ds(0, 512)],
                            acc_sh.at[pl.ds(15 * _STRIPE, 512)])

        pltpu.sync_copy(src_hbm.at[wid], src_v)
        pltpu.sync_copy(dst_hbm.at[wid], dst_v)
        plsc.subcore_barrier()

        pltpu.async_copy(tab_hbm.at[src_v.at[0]], rows.at[0], sem_g[0])
        pltpu.async_copy(tab_hbm.at[src_v.at[1]], rows.at[1], sem_g[1])

        def body(t, carry):
            j0 = 2 * t
            for k2 in range(2):
                pltpu.make_async_copy(tab_hbm.at[src_v.at[j0 + k2]],
                                      rows.at[k2], sem_g[k2]).wait()
                pltpu.sync_copy(rows.at[k2], acc_sh.at[dst_v.at[j0 + k2]],
                                add=True)

                @pl.when(j0 + 2 + k2 < nch)
                def _():
                    pltpu.async_copy(tab_hbm.at[src_v.at[j0 + 2 + k2]],
                                     rows.at[k2], sem_g[k2])
            return carry

        lax.fori_loop(0, nch // 2, body, 0)
        plsc.subcore_barrier()

        @pl.when(s < 15)
        def _():
            pltpu.sync_copy(acc_sh.at[pl.ds(s * _STRIPE, _STRIPE)],
                            out_hbm.at[c].at[pl.ds(s * _STRIPE, _STRIPE)])

        @pl.when(s == 15)
        def _():
            pltpu.sync_copy(acc_sh.at[pl.ds(15 * _STRIPE, 512)],
                            out_hbm.at[c].at[pl.ds(15 * _STRIPE, 512)])

    return k


def _pad_idx(idx, fill, nch, chunk):
    pad = _NW * nch * chunk - _E
    fidx = jnp.arange(pad, dtype=jnp.int32)
    return jnp.concatenate([idx, fill(fidx)]).reshape(_NW, nch, chunk)


def kernel(x, edge_index, batch_sizes, att_W, att_b, feat_W, feat_b,
           glb_W, glb_b, sage_Wl, sage_bl, sage_Wr):
    offs = jnp.concatenate([jnp.zeros((1,), jnp.int32),
                            jnp.cumsum(batch_sizes, dtype=jnp.int32)])
    lo = offs[:_B].reshape(1, _B)
    hi = offs[1:].reshape(1, _B)
    x_pad = jnp.concatenate(
        [x, jnp.zeros((_NPAD - _N, _D), jnp.float32)], axis=0)
    src = edge_index[0]
    dst = edge_index[1]
    gid = (jnp.arange(_N, dtype=jnp.int32)[:, None]
           >= offs[1:][None, :]).sum(axis=1, dtype=jnp.int32)
    gsrc = gid[src]
    src_p = _pad_idx(src, lambda f: f % _N, _NCH, _CHUNK)
    dst_p = _pad_idx(dst, lambda f: _N + f % (_ACC - _N), _NCH, _CHUNK)
    gsrc_p = _pad_idx(gsrc, lambda f: jnp.zeros_like(f), _NCH, _CHUNK)
    zer = jnp.zeros((_STRIPE, _H), jnp.float32)
    eye = jnp.eye(128, dtype=jnp.float32)
    eyeA = jnp.concatenate([eye, jnp.zeros((16, 128), jnp.float32)], axis=0)
    eyeB = jnp.concatenate([jnp.zeros((128, 128), jnp.float32),
                            eye[: _B - 128],
                            jnp.zeros((144 - _B, 128), jnp.float32)], axis=0)
    g = jnp.zeros((_B, _H), jnp.float32)

    aw = lambda i: att_W[i]
    ab = lambda i: att_b[i].reshape(1, _H)
    fw = lambda i: feat_W[i]
    fb = lambda i: feat_b[i].reshape(1, _H)
    gw = lambda i: glb_W[i]
    gb = lambda i: glb_b[i].reshape(1, _H)
    wlt = lambda i: sage_Wl[i, :_D]
    wlb = lambda i: sage_Wl[i, _D:]
    wr = lambda i: sage_Wr[i]
    blr = lambda i: sage_bl[i].reshape(1, _H)

    cpa = _sc_scatter(_H, _CHUNK, _NCH)(eyeA, gsrc_p, dst_p, zer)
    cpb = _sc_scatter(_H, _CHUNK, _NCH)(eyeB, gsrc_p, dst_p, zer)
    Ca = _csum()(cpa)
    Cb = _csum()(cpb)
    aggs = _sc_scatter(_H, _CHUNK, _NCH)(x_pad, src_p, dst_p, zer)
    out = _p1_first()(x_pad, lo, hi, aw(0), ab(0), fw(0), fb(0))
    g = _p3p()(out, g, gw(0), gb(0))
    q, aggg = _p3q()(x_pad, lo, hi, g, wr(0), Ca, Cb)
    for i in (1, 2):
        h, out = _p1_mid()(aggs, aggg, q, blr(i - 1), lo, hi,
                           wlt(i - 1), wlb(i - 1), aw(i), ab(i), fw(i),
                           fb(i))
        aggs = _sc_scatter(_H, _CHUNK, _NCH)(h, src_p, dst_p, zer)
        g = _p3p()(out, g, gw(i), gb(i))
        q, aggg = _p3q()(h, lo, hi, g, wr(i), Ca, Cb)
    h, _, g = _p1_last()(aggs, aggg, q, blr(2), lo, hi, g, wlt(2), wlb(2),
                         aw(3), ab(3), fw(3), fb(3), gw(3), gb(3))
    return (h[:_N], g)


# speedup vs baseline: 19.2177x; 3.8042x over previous
import functools

import jax
import jax.numpy as jnp
from jax import lax
from jax.experimental import pallas as pl
from jax.experimental.pallas import tpu as pltpu
from jax.experimental.pallas import tpu_sc as plsc

_N = 9870
_D = 128
_H = 128
_B = 141
_E = 157920

_NW = 32
_CHUNK = 128
_NCH = 40
_STRIPE = 624
_ACC = 15 * _STRIPE + 512
_NPAD = 9984

_R = 1248
_NB = _NPAD // _R

_HI = lax.Precision.HIGHEST


def _dot(a, b):
    return jnp.dot(a, b, preferred_element_type=jnp.float32, precision=_HI)


def _dot_d(a, b):
    return jnp.dot(a, b, preferred_element_type=jnp.float32)


def _onehot(lo, hi, j):
    r = (j * _R + lax.broadcasted_iota(jnp.int32, (_R, _B), 0))
    return ((r >= lo) & (r < hi)).astype(jnp.float32)


def _softmax(al):
    m = jnp.max(al, axis=-1, keepdims=True)
    e = jnp.exp(al - m)
    return e / jnp.sum(e, axis=-1, keepdims=True)


def _gnew_of(out, g, glbW, glbb):
    z = jnp.concatenate([out, g], axis=1)
    return g + jnp.tanh(_dot_d(z, glbW) + glbb)


def _relu_rows(aggh_ref, aggg_ref, q_ref, bl_ref, wlt, wlb):
    agg_h = aggh_ref[0] + aggh_ref[1]
    h = jnp.maximum(_dot_d(agg_h, wlt) + _dot_d(aggg_ref[...], wlb)
                    + bl_ref[...] + q_ref[...], 0.0)
    r = (pl.program_id(0) * _R
         + lax.broadcasted_iota(jnp.int32, (_R, 1), 0))
    return jnp.where(r < _N, h, 0.0)


def _p1_body(first, last):
    def body(*refs):
        if first:
            (x_ref, lo_ref, hi_ref, attW, attb, featW, featb, out_ref) = refs
            h = x_ref[...]
        elif last:
            (aggh_ref, aggg_ref, q_ref, bl_ref, lo_ref, hi_ref, g_ref,
             wlt, wlb, attW, attb, featW, featb, glbW, glbb,
             h_out, out_ref, g_out) = refs
            h = _relu_rows(aggh_ref, aggg_ref, q_ref, bl_ref, wlt[...],
                           wlb[...])
            h_out[...] = h
        else:
            (aggh_ref, aggg_ref, q_ref, bl_ref, lo_ref, hi_ref,
             wlt, wlb, attW, attb, featW, featb, h_out, out_ref) = refs
            h = _relu_rows(aggh_ref, aggg_ref, q_ref, bl_ref, wlt[...],
                           wlb[...])
            h_out[...] = h
        j = pl.program_id(0)
        a = _softmax(_dot_d(h, attW[...]) + attb[...])
        f = _dot_d(h, featW[...]) + featb[...]
        prod = a * f
        oh = _onehot(lo_ref[...], hi_ref[...], j)
        contrib = lax.dot_general(oh, prod, (((0,), (0,)), ((), ())),
                                  preferred_element_type=jnp.float32)

        @pl.when(j == 0)
        def _():
            out_ref[...] = jnp.zeros_like(out_ref)

        out_ref[...] += contrib
        if last:
            @pl.when(j == _NB - 1)
            def _():
                g_out[...] = _gnew_of(out_ref[...], g_ref[...], glbW[...],
                                      glbb[...])
    return body


def _p3p_body(out_ref, g_ref, glbW, glbb, g_out):
    g_out[...] = _gnew_of(out_ref[...], g_ref[...], glbW[...], glbb[...])


def _p3q_body(h_ref, lo_ref, hi_ref, g_ref, wr, Ca_ref, Cb_ref, q_out,
              aggg_out):
    j = pl.program_id(0)
    oh = _onehot(lo_ref[...], hi_ref[...], j)
    g = g_ref[...]
    grep = _dot(oh, g)
    h2 = jnp.concatenate([h_ref[...], grep], axis=1)
    q_out[...] = _dot_d(h2, wr[...])
    ga = g[0:128, :]
    gb = jnp.concatenate(
        [g[128:_B, :], jnp.zeros((128 - (_B - 128), _H), jnp.float32)],
        axis=0)
    aggg_out[...] = _dot(Ca_ref[...], ga) + _dot(Cb_ref[...], gb)


def _csum_body(cp_ref, c_out):
    c_out[...] = cp_ref[0] + cp_ref[1]


_row = pl.BlockSpec((_R, _H), lambda j: (j, 0))
_full = lambda r, c: pl.BlockSpec((r, c), lambda j: (0, 0))
_segb = pl.BlockSpec((1, _B), lambda j: (0, 0))
_aggb = pl.BlockSpec((2, _R, _H), lambda j: (0, j, 0))
_cb = pl.BlockSpec((_R, _H), lambda j: (j, 0))
_cpb = pl.BlockSpec((2, _R, _H), lambda j: (0, j, 0))
_w = _full(_H, _H)
_w2 = _full(2 * _H, _H)
_b1 = _full(1, _H)
_gB = _full(_B, _H)

_f32 = jnp.float32


def _p1_first():
    return pl.pallas_call(
        _p1_body(True, False),
        grid=(_NB,),
        in_specs=[_row, _segb, _segb, _w, _b1, _w, _b1],
        out_specs=_gB,
        out_shape=jax.ShapeDtypeStruct((_B, _H), _f32),
    )


def _p1_mid():
    return pl.pallas_call(
        _p1_body(False, False),
        grid=(_NB,),
        in_specs=[_aggb, _row, _row, _b1, _segb, _segb, _w, _w, _w, _b1,
                  _w, _b1],
        out_specs=(_row, _gB),
        out_shape=(jax.ShapeDtypeStruct((_NPAD, _H), _f32),
                   jax.ShapeDtypeStruct((_B, _H), _f32)),
    )


def _p1_last():
    return pl.pallas_call(
        _p1_body(False, True),
        grid=(_NB,),
        in_specs=[_aggb, _row, _row, _b1, _segb, _segb, _gB, _w, _w,
                  _w, _b1, _w, _b1, _w2, _b1],
        out_specs=(_row, _gB, _gB),
        out_shape=(jax.ShapeDtypeStruct((_NPAD, _H), _f32),
                   jax.ShapeDtypeStruct((_B, _H), _f32),
                   jax.ShapeDtypeStruct((_B, _H), _f32)),
    )


def _p3p():
    return pl.pallas_call(
        _p3p_body,
        out_shape=jax.ShapeDtypeStruct((_B, _H), _f32),
    )


def _p3q():
    return pl.pallas_call(
        _p3q_body,
        grid=(_NB,),
        in_specs=[_row, _segb, _segb, _gB, _w2, _cb, _cb],
        out_specs=(_row, _row),
        out_shape=(jax.ShapeDtypeStruct((_NPAD, _H), _f32),
                   jax.ShapeDtypeStruct((_NPAD, _H), _f32)),
    )


def _csum():
    return pl.pallas_call(
        _csum_body,
        grid=(_NB,),
        in_specs=[_cpb],
        out_specs=_cb,
        out_shape=jax.ShapeDtypeStruct((_ACC, _H), _f32),
    )


def _sc_scatter(width, chunk, nch):
    @functools.partial(
        pl.kernel,
        out_type=jax.ShapeDtypeStruct((2, _ACC, width), jnp.float32),
        mesh=plsc.VectorSubcoreMesh(core_axis_name="c", subcore_axis_name="s"),
        scratch_types=[
            pltpu.VMEM((nch, chunk), jnp.int32),
            pltpu.VMEM((nch, chunk), jnp.int32),
            pltpu.VMEM((2, chunk, width), jnp.float32),
            pltpu.VMEM_SHARED((_ACC, width), jnp.float32),
            pltpu.SemaphoreType.DMA,
            pltpu.SemaphoreType.DMA,
        ],
    )
    def k(tab_hbm, src_hbm, dst_hbm, zer_hbm, out_hbm, src_v, dst_v, rows,
          acc_sh, sg0, sg1):
        sem_g = [sg0, sg1]
        c = lax.axis_index("c")
        s = lax.axis_index("s")
        wid = s * 2 + c

        @pl.when(s < 15)
        def _():
            pltpu.sync_copy(zer_hbm, acc_sh.at[pl.ds(s * _STRIPE, _STRIPE)])

        @pl.when(s == 15)
        def _():
            pltpu.sync_copy(zer_hbm.at[pl.ds(0, 512)],
                            acc_sh.at[pl.ds(15 * _STRIPE, 512)])

        pltpu.sync_copy(src_hbm.at[wid], src_v)
        pltpu.sync_copy(dst_hbm.at[wid], dst_v)
        plsc.subcore_barrier()

        pltpu.async_copy(tab_hbm.at[src_v.at[0]], rows.at[0], sem_g[0])
        pltpu.async_copy(tab_hbm.at[src_v.at[1]], rows.at[1], sem_g[1])

        def body(t, carry):
            j0 = 2 * t
            for k2 in range(2):
                pltpu.make_async_copy(tab_hbm.at[src_v.at[j0 + k2]],
                                      rows.at[k2], sem_g[k2]).wait()
                pltpu.sync_copy(rows.at[k2], acc_sh.at[dst_v.at[j0 + k2]],
                                add=True)

                @pl.when(j0 + 2 + k2 < nch)
                def _():
                    pltpu.async_copy(tab_hbm.at[src_v.at[j0 + 2 + k2]],
                                     rows.at[k2], sem_g[k2])
            return carry

        lax.fori_loop(0, nch // 2, body, 0)
        plsc.subcore_barrier()

        @pl.when(s < 15)
        def _():
            pltpu.sync_copy(acc_sh.at[pl.ds(s * _STRIPE, _STRIPE)],
                            out_hbm.at[c].at[pl.ds(s * _STRIPE, _STRIPE)])

        @pl.when(s == 15)
        def _():
            pltpu.sync_copy(acc_sh.at[pl.ds(15 * _STRIPE, 512)],
                            out_hbm.at[c].at[pl.ds(15 * _STRIPE, 512)])

    return k


def _pad_idx(idx, fill, nch, chunk):
    pad = _NW * nch * chunk - _E
    fidx = jnp.arange(pad, dtype=jnp.int32)
    return jnp.concatenate([idx, fill(fidx)]).reshape(_NW, nch, chunk)


def kernel(x, edge_index, batch_sizes, att_W, att_b, feat_W, feat_b,
           glb_W, glb_b, sage_Wl, sage_bl, sage_Wr):
    offs = jnp.concatenate([jnp.zeros((1,), jnp.int32),
                            jnp.cumsum(batch_sizes, dtype=jnp.int32)])
    lo = offs[:_B].reshape(1, _B)
    hi = offs[1:].reshape(1, _B)
    x_pad = jnp.concatenate(
        [x, jnp.zeros((_NPAD - _N, _D), jnp.float32)], axis=0)
    src = edge_index[0]
    dst = edge_index[1]
    rn = jnp.arange(_NPAD, dtype=jnp.int32)[:, None]
    ohA = ((rn >= offs[:128][None, :])
           & (rn < offs[1:129][None, :])).astype(jnp.float32)
    ohB = jnp.pad(((rn >= offs[128:_B][None, :])
                   & (rn < offs[129:_B + 1][None, :])).astype(jnp.float32),
                  ((0, 0), (0, 128 - (_B - 128))))
    src_p = _pad_idx(src, lambda f: f % _N, _NCH, _CHUNK)
    dst_p = _pad_idx(dst, lambda f: _N + f % (_ACC - _N), _NCH, _CHUNK)
    zer = jnp.zeros((_STRIPE, _H), jnp.float32)
    g = jnp.zeros((_B, _H), jnp.float32)

    aw = lambda i: att_W[i]
    ab = lambda i: att_b[i].reshape(1, _H)
    fw = lambda i: feat_W[i]
    fb = lambda i: feat_b[i].reshape(1, _H)
    gw = lambda i: glb_W[i]
    gb = lambda i: glb_b[i].reshape(1, _H)
    wlt = lambda i: sage_Wl[i, :_D]
    wlb = lambda i: sage_Wl[i, _D:]
    wr = lambda i: sage_Wr[i]
    blr = lambda i: sage_bl[i].reshape(1, _H)

    cpa = _sc_scatter(_H, _CHUNK, _NCH)(ohA, src_p, dst_p, zer)
    cpb = _sc_scatter(_H, _CHUNK, _NCH)(ohB, src_p, dst_p, zer)
    Ca = _csum()(cpa)
    Cb = _csum()(cpb)
    aggs = _sc_scatter(_H, _CHUNK, _NCH)(x_pad, src_p, dst_p, zer)
    out = _p1_first()(x_pad, lo, hi, aw(0), ab(0), fw(0), fb(0))
    g = _p3p()(out, g, gw(0), gb(0))
    q, aggg = _p3q()(x_pad, lo, hi, g, wr(0), Ca, Cb)
    for i in (1, 2):
        h, out = _p1_mid()(aggs, aggg, q, blr(i - 1), lo, hi,
                           wlt(i - 1), wlb(i - 1), aw(i), ab(i), fw(i),
                           fb(i))
        aggs = _sc_scatter(_H, _CHUNK, _NCH)(h, src_p, dst_p, zer)
        g = _p3p()(out, g, gw(i), gb(i))
        q, aggg = _p3q()(h, lo, hi, g, wr(i), Ca, Cb)
    h, _, g = _p1_last()(aggs, aggg, q, blr(2), lo, hi, g, wlt(2), wlb(2),
                         aw(3), ab(3), fw(3), fb(3), gw(3), gb(3))
    return (h[:_N], g)
